# baseline (speedup 1.0000x reference)
import functools

import jax
import jax.numpy as jnp
from jax import lax
from jax.experimental import pallas as pl
from jax.experimental.pallas import tpu as pltpu
from jax.experimental.pallas import tpu_sc as plsc

N = 10000
E = 320000
D = 128
DH = D // 2
ALPHA = 0.2

NC, NS, L = 2, 16, 16
NPAD = 10240
HPK = NPAD // 2
SPAD = 10048
ROWS_PER_TILE = HPK // NS
EPT = E // NS
EPTP = 20480
CHUNK = 32
NCHUNKS = EPTP // CHUNK
BLK = 16
NBLK = NCHUNKS // BLK
BPAIRS = BLK // 2 - 1


def _tc_prep(x_ref, w_ref, a_ref, h_ref, s1_ref, s2_ref, c_ref):
    h = jnp.dot(x_ref[...], w_ref[...], preferred_element_type=jnp.float32)
    h_ref[...] = h
    a = a_ref[...]
    s1 = jnp.sum(h * a[:D, 0][None, :], axis=1)
    s2 = jnp.sum(h * a[D:, 0][None, :], axis=1)
    s1_ref[...] = s1
    s2_ref[...] = s2
    m = jnp.max(s1) + jnp.max(s2)
    c_ref[...] = jnp.full((16,), jnp.where(m >= 0.0, m, ALPHA * m),
                          dtype=jnp.float32)


def _sc_edges(hpk_hbm, row_hbm, col_hbm, s1_hbm, s2_hbm, c_hbm,
              znd_hbm, zn_hbm, u_out, rs_out,
              s1_v, s2_v, c_v, row_t, col_t, rsh_t, csh_t,
              wb_t, rpf_t, rows_a, rows_b, stage_v,
              u_sh, h_sh, rs_sh, sem_a, sem_b):
    cid = lax.axis_index("c")
    sid = lax.axis_index("s")
    pltpu.sync_copy(s1_hbm, s1_v)
    pltpu.sync_copy(s2_hbm, s2_v)
    pltpu.sync_copy(c_hbm, c_v)
    r0 = sid * ROWS_PER_TILE
    pltpu.sync_copy(hpk_hbm.at[cid, pl.ds(r0, ROWS_PER_TILE)],
                    h_sh.at[pl.ds(r0, ROWS_PER_TILE)])
    pltpu.sync_copy(znd_hbm.at[pl.ds(r0, ROWS_PER_TILE)],
                    u_sh.at[pl.ds(r0, ROWS_PER_TILE)])
    n0 = sid * (NPAD // NS)
    pltpu.sync_copy(zn_hbm.at[pl.ds(n0, NPAD // NS)],
                    rs_sh.at[pl.ds(n0, NPAD // NS)])
    plsc.subcore_barrier()

    cvec = c_v[pl.ds(0, L)]
    iota = lax.iota(jnp.int32, L)

    def issue_gather(g, buf, sem):
        pltpu.async_copy(h_sh.at[csh_t.at[g]], buf, sem)

    def wait_gather(g, buf, sem):
        pltpu.make_async_copy(h_sh.at[csh_t.at[g]], buf, sem).wait()

    def prep_chunk(g, carry):
        for i in range(CHUNK // L):
            sl = pl.ds(i * L, L)
            idxr = row_t[g, sl]
            idxc = col_t[g, sl]
            e = plsc.load_gather(s1_v, [idxr]) + plsc.load_gather(s2_v, [idxc])
            e = jnp.where(e >= 0.0, e, ALPHA * e)
            wb_t[g, sl] = jnp.exp(e - cvec)
            rsh_t[g, sl] = idxr >> 1
            csh_t[g, sl] = idxc >> 1
            rpf_t[g, sl] = (idxr & 1).astype(jnp.float32)
        return carry

    def process(g, buf):
        def scale_body(ei, c2):
            ei16 = jnp.full((L,), ei, jnp.int32)
            gvec = jnp.full((L,), g, jnp.int32)
            w16 = plsc.load_gather(wb_t, [gvec, ei16])
            rp16 = plsc.load_gather(rpf_t, [gvec, ei16])
            cp16 = plsc.load_gather(col_t, [gvec, ei16]) & 1
            whi = w16 * rp16
            wlo = w16 - whi
            cbase = cp16 * DH + iota
            for j in range(DH // L):
                src = plsc.load_gather(buf, [ei16, cbase + (j * L)])
                stage_v[ei, pl.ds(j * L, L)] = src * wlo
                stage_v[ei, pl.ds(DH + j * L, L)] = src * whi
            return c2

        lax.fori_loop(0, CHUNK, scale_body, 0)
        pltpu.sync_copy(stage_v, u_sh.at[rsh_t.at[g]], add=True)
        pltpu.sync_copy(wb_t.at[g], rs_sh.at[row_t.at[g]], add=True)

    def block_body(b, carry):
        boff = pl.multiple_of(b * BLK, 8)
        pltpu.sync_copy(row_hbm.at[sid, pl.ds(boff, BLK)], row_t)
        pltpu.sync_copy(col_hbm.at[sid, pl.ds(boff, BLK)], col_t)
        lax.fori_loop(0, BLK, prep_chunk, 0)
        issue_gather(0, rows_a, sem_a)

        def pair_body(p, c2):
            a = 2 * p
            issue_gather(a + 1, rows_b, sem_b)
            wait_gather(a, rows_a, sem_a)
            process(a, rows_a)
            issue_gather(a + 2, rows_a, sem_a)
            wait_gather(a + 1, rows_b, sem_b)
            process(a + 1, rows_b)
            return c2

        lax.fori_loop(0, BPAIRS, pair_body, 0)
        issue_gather(BLK - 1, rows_b, sem_b)
        wait_gather(BLK - 2, rows_a, sem_a)
        process(BLK - 2, rows_a)
        wait_gather(BLK - 1, rows_b, sem_b)
        process(BLK - 1, rows_b)
        return carry

    lax.fori_loop(0, NBLK, block_body, 0)

    plsc.subcore_barrier()
    pltpu.sync_copy(u_sh.at[pl.ds(r0, ROWS_PER_TILE)],
                    u_out.at[cid, pl.ds(r0, ROWS_PER_TILE)])
    pltpu.sync_copy(rs_sh.at[pl.ds(n0, NPAD // NS)],
                    rs_out.at[cid, pl.ds(n0, NPAD // NS)])


_sc_edges_call = functools.partial(
    pl.kernel,
    out_type=[jax.ShapeDtypeStruct((NC, HPK, D), jnp.float32),
              jax.ShapeDtypeStruct((NC, NPAD), jnp.float32)],
    mesh=plsc.VectorSubcoreMesh(core_axis_name="c", subcore_axis_name="s"),
    compiler_params=pltpu.CompilerParams(needs_layout_passes=False),
    scratch_types=[
        pltpu.VMEM((SPAD,), jnp.float32),
        pltpu.VMEM((SPAD,), jnp.float32),
        pltpu.VMEM((16,), jnp.float32),
        pltpu.VMEM((BLK, CHUNK), jnp.int32),
        pltpu.VMEM((BLK, CHUNK), jnp.int32),
        pltpu.VMEM((BLK, CHUNK), jnp.int32),
        pltpu.VMEM((BLK, CHUNK), jnp.int32),
        pltpu.VMEM((BLK, CHUNK), jnp.float32),
        pltpu.VMEM((BLK, CHUNK), jnp.float32),
        pltpu.VMEM((CHUNK, D), jnp.float32),
        pltpu.VMEM((CHUNK, D), jnp.float32),
        pltpu.VMEM((CHUNK, D), jnp.float32),
        pltpu.VMEM_SHARED((HPK, D), jnp.float32),
        pltpu.VMEM_SHARED((HPK, D), jnp.float32),
        pltpu.VMEM_SHARED((NPAD,), jnp.float32),
        pltpu.SemaphoreType.DMA,
        pltpu.SemaphoreType.DMA,
    ],
)(_sc_edges)


def _tc_final(u_ref, rs_ref, o_ref):
    u = u_ref[...]
    rs = jnp.clip(rs_ref[0][:, None], 1e-8, None)
    hp = u / rs
    o_ref[...] = jnp.where(hp > 0.0, hp, jnp.exp(jnp.minimum(hp, 0.0)) - 1.0)


def kernel(x, edge_index, W, a):
    h, s1, s2, c = pl.pallas_call(
        _tc_prep,
        out_shape=[
            jax.ShapeDtypeStruct((N, D), jnp.float32),
            jax.ShapeDtypeStruct((N,), jnp.float32),
            jax.ShapeDtypeStruct((N,), jnp.float32),
            jax.ShapeDtypeStruct((16,), jnp.float32),
        ],
    )(x, W, a)
    hpd = jnp.pad(h, ((0, NPAD - N), (0, 0)))
    hpk = jnp.stack([hpd[:, :DH].reshape(HPK, D),
                     hpd[:, DH:].reshape(HPK, D)])
    row = edge_index[0].reshape(NS, EPT)
    col = edge_index[1].reshape(NS, EPT)
    rpad = jnp.full((NS, EPTP - EPT), N, jnp.int32)
    cpad = jnp.zeros((NS, EPTP - EPT), jnp.int32)
    row = jnp.concatenate([row, rpad], axis=1).reshape(NS, NCHUNKS, CHUNK)
    col = jnp.concatenate([col, cpad], axis=1).reshape(NS, NCHUNKS, CHUNK)
    s1 = jnp.pad(s1, (0, SPAD - N))
    s2 = jnp.pad(s2, (0, SPAD - N))
    znd = jnp.zeros((HPK, D), jnp.float32)
    zn = jnp.zeros((NPAD,), jnp.float32)
    u_parts, rs_parts = _sc_edges_call(hpk, row, col, s1, s2, c, znd, zn)
    u_full = jnp.concatenate([u_parts[0].reshape(NPAD, DH),
                              u_parts[1].reshape(NPAD, DH)], axis=1)
    out = pl.pallas_call(
        _tc_final,
        out_shape=jax.ShapeDtypeStruct((NPAD, D), jnp.float32),
    )(u_full, rs_parts)
    return out[:N]


# speedup vs baseline: 2.1231x; 2.1231x over previous
import functools

import jax
import jax.numpy as jnp
from jax import lax
from jax.experimental import pallas as pl
from jax.experimental.pallas import tpu as pltpu
from jax.experimental.pallas import tpu_sc as plsc

N = 10000
E = 320000
D = 128
ALPHA = 0.2

NC, NS, L = 2, 16, 16
NW = NC * NS
NPAD = 10240
ROWS_PER_TILE = NPAD // NS
EPW = E // NW
CHUNK = 80
NCHUNKS = EPW // CHUNK


def _tc_prep(x_ref, w_ref, a_ref, h_ref, s1_ref, s2_ref, c_ref):
    h = jnp.dot(x_ref[...], w_ref[...], preferred_element_type=jnp.float32)
    h_ref[...] = h
    a = a_ref[...]
    s1 = jnp.sum(h * a[:D, 0][None, :], axis=1)
    s2 = jnp.sum(h * a[D:, 0][None, :], axis=1)
    s1_ref[...] = s1
    s2_ref[...] = s2
    m = jnp.max(s1) + jnp.max(s2)
    c_ref[...] = jnp.full((16,), jnp.where(m >= 0.0, m, ALPHA * m),
                          dtype=jnp.float32)


def _sc_edges(h_hbm, row_hbm, col_hbm, s1_hbm, s2_hbm, c_hbm,
              znd_hbm, zn_hbm, u_out, rs_out,
              s1_v, s2_v, c_v, row_v, col_v, rows_v, w_v,
              u_sh, rs_sh, sem):
    cid = lax.axis_index("c")
    sid = lax.axis_index("s")
    wid = cid * NS + sid
    pltpu.sync_copy(s1_hbm, s1_v)
    pltpu.sync_copy(s2_hbm, s2_v)
    pltpu.sync_copy(c_hbm, c_v)
    r0 = sid * ROWS_PER_TILE
    pltpu.sync_copy(znd_hbm.at[pl.ds(r0, ROWS_PER_TILE)],
                    u_sh.at[pl.ds(r0, ROWS_PER_TILE)])
    pltpu.sync_copy(zn_hbm.at[pl.ds(r0, ROWS_PER_TILE)],
                    rs_sh.at[pl.ds(r0, ROWS_PER_TILE)])
    plsc.subcore_barrier()

    cvec = c_v[pl.ds(0, L)]
    base = wid * EPW

    def chunk_body(g, carry):
        off = base + g * CHUNK
        pltpu.sync_copy(row_hbm.at[pl.ds(off, CHUNK)], row_v)
        pltpu.sync_copy(col_hbm.at[pl.ds(off, CHUNK)], col_v)
        pltpu.async_copy(h_hbm.at[col_v], rows_v, sem).wait()
        for i in range(CHUNK // L):
            idxr = row_v[pl.ds(i * L, L)]
            idxc = col_v[pl.ds(i * L, L)]
            e = plsc.load_gather(s1_v, [idxr]) + plsc.load_gather(s2_v, [idxc])
            e = jnp.where(e >= 0.0, e, ALPHA * e)
            w_v[pl.ds(i * L, L)] = jnp.exp(e - cvec)

        def scale_body(ei, c2):
            ws = plsc.load_gather(w_v, [jnp.full((L,), ei, jnp.int32)])
            for j in range(D // L):
                rows_v[ei, pl.ds(j * L, L)] = rows_v[ei, pl.ds(j * L, L)] * ws
            return c2

        lax.fori_loop(0, CHUNK, scale_body, 0)
        pltpu.sync_copy(rows_v, u_sh.at[row_v], add=True)
        pltpu.sync_copy(w_v, rs_sh.at[row_v], add=True)
        return carry

    lax.fori_loop(0, NCHUNKS, chunk_body, 0)
    plsc.subcore_barrier()
    pltpu.sync_copy(u_sh.at[pl.ds(r0, ROWS_PER_TILE)],
                    u_out.at[cid, pl.ds(r0, ROWS_PER_TILE)])
    pltpu.sync_copy(rs_sh.at[pl.ds(r0, ROWS_PER_TILE)],
                    rs_out.at[cid, pl.ds(r0, ROWS_PER_TILE)])


_sc_edges_call = functools.partial(
    pl.kernel,
    out_type=[jax.ShapeDtypeStruct((NC, NPAD, D), jnp.float32),
              jax.ShapeDtypeStruct((NC, NPAD), jnp.float32)],
    mesh=plsc.VectorSubcoreMesh(core_axis_name="c", subcore_axis_name="s"),
    compiler_params=pltpu.CompilerParams(needs_layout_passes=False),
    scratch_types=[
        pltpu.VMEM((N,), jnp.float32),
        pltpu.VMEM((N,), jnp.float32),
        pltpu.VMEM((16,), jnp.float32),
        pltpu.VMEM((CHUNK,), jnp.int32),
        pltpu.VMEM((CHUNK,), jnp.int32),
        pltpu.VMEM((CHUNK, D), jnp.float32),
        pltpu.VMEM((CHUNK,), jnp.float32),
        pltpu.VMEM_SHARED((NPAD, D), jnp.float32),
        pltpu.VMEM_SHARED((NPAD,), jnp.float32),
        pltpu.SemaphoreType.DMA,
    ],
)(_sc_edges)


def _tc_final(u_ref, rs_ref, o_ref):
    u = u_ref[0] + u_ref[1]
    rs = jnp.clip(rs_ref[0] + rs_ref[1], 1e-8, None)
    hp = u / rs[:, None]
    o_ref[...] = jnp.where(hp > 0.0, hp, jnp.exp(jnp.minimum(hp, 0.0)) - 1.0)


def kernel(x, edge_index, W, a):
    h, s1, s2, c = pl.pallas_call(
        _tc_prep,
        out_shape=[
            jax.ShapeDtypeStruct((N, D), jnp.float32),
            jax.ShapeDtypeStruct((N,), jnp.float32),
            jax.ShapeDtypeStruct((N,), jnp.float32),
            jax.ShapeDtypeStruct((16,), jnp.float32),
        ],
    )(x, W, a)
    row = edge_index[0]
    col = edge_index[1]
    znd = jnp.zeros((NPAD, D), jnp.float32)
    zn = jnp.zeros((NPAD,), jnp.float32)
    u_parts, rs_parts = _sc_edges_call(h, row, col, s1, s2, c, znd, zn)
    out = pl.pallas_call(
        _tc_final,
        out_shape=jax.ShapeDtypeStruct((NPAD, D), jnp.float32),
    )(u_parts, rs_parts)
    return out[:N]
